# Initial kernel scaffold; baseline (speedup 1.0000x reference)
import jax
import jax.numpy as jnp
from jax.experimental import pallas as pl


def kernel(edges, shape_features, Ws_self, Ws_neigh, bs, Wf_self, Wf_neigh, bf):
    raise NotImplementedError("write your pallas kernel here")


# speedup vs baseline: 1.0547x; 1.0547x over previous
import functools

import jax
import jax.numpy as jnp
from jax.experimental import pallas as pl
from jax.experimental.pallas import tpu as pltpu

N = 10000
D = 128
BLK = 2000


def _dense_body(x_ref, agg_ref, ws_ref, wn_ref, b_ref, o_ref, *, mode):
    acc = jnp.dot(x_ref[...], ws_ref[...], preferred_element_type=jnp.float32)
    acc += jnp.dot(agg_ref[...], wn_ref[...], preferred_element_type=jnp.float32)
    acc += b_ref[...]
    if mode == "relu":
        o_ref[...] = jnp.maximum(acc, 0.0)
    elif mode == "linear":
        o_ref[...] = acc
    else:
        raise ValueError(mode)


def _dense_res_body(x_ref, agg_ref, ws_ref, wn_ref, b_ref, temp_ref, o_ref):
    acc = jnp.dot(x_ref[...], ws_ref[...], preferred_element_type=jnp.float32)
    acc += jnp.dot(agg_ref[...], wn_ref[...], preferred_element_type=jnp.float32)
    acc += b_ref[...]
    o_ref[...] = (temp_ref[...] + jnp.maximum(acc, 0.0)) * 0.5


def _row_spec():
    return pl.BlockSpec((BLK, D), lambda i: (i, 0))


def _w_spec():
    return pl.BlockSpec((D, D), lambda i: (0, 0))


def _b_spec():
    return pl.BlockSpec((1, D), lambda i: (0, 0))


@functools.partial(jax.jit, static_argnames=("mode",))
def _dense_layer(x, agg, ws, wn, b, mode):
    return pl.pallas_call(
        functools.partial(_dense_body, mode=mode),
        grid=(N // BLK,),
        in_specs=[_row_spec(), _row_spec(), _w_spec(), _w_spec(), _b_spec()],
        out_specs=_row_spec(),
        out_shape=jax.ShapeDtypeStruct((N, D), jnp.float32),
    )(x, agg, ws, wn, b.reshape(1, D))


@jax.jit
def _dense_res_layer(x, agg, ws, wn, b, temp):
    return pl.pallas_call(
        _dense_res_body,
        grid=(N // BLK,),
        in_specs=[_row_spec(), _row_spec(), _w_spec(), _w_spec(), _b_spec(),
                  _row_spec()],
        out_specs=_row_spec(),
        out_shape=jax.ShapeDtypeStruct((N, D), jnp.float32),
    )(x, agg, ws, wn, b.reshape(1, D), temp)


def _segsum(x, src, dst):
    gathered = jnp.take(x, src, axis=0)
    return jax.ops.segment_sum(gathered, dst, num_segments=N)


def kernel(edges, shape_features, Ws_self, Ws_neigh, bs, Wf_self, Wf_neigh, bf):
    src = edges[0]
    dst = edges[1]

    x = _dense_layer(shape_features, _segsum(shape_features, src, dst),
                     Ws_self[0], Ws_neigh[0], bs[0], mode="relu")
    for i in range(1, 12, 2):
        temp = x
        x = _dense_layer(x, _segsum(x, src, dst),
                         Ws_self[i], Ws_neigh[i], bs[i], mode="relu")
        x = _dense_res_layer(x, _segsum(x, src, dst),
                             Ws_self[i + 1], Ws_neigh[i + 1], bs[i + 1], temp)

    wf_s = jnp.zeros((D, D), jnp.float32).at[:, :3].set(Wf_self)
    wf_n = jnp.zeros((D, D), jnp.float32).at[:, :3].set(Wf_neigh)
    bf_p = jnp.zeros((D,), jnp.float32).at[:3].set(bf)
    coords_p = _dense_layer(x, _segsum(x, src, dst), wf_s, wf_n, bf_p,
                            mode="linear")
    return (x, coords_p[:, :3])


# speedup vs baseline: 2.5109x; 2.3807x over previous
import functools

import jax
import jax.numpy as jnp
from jax import lax
from jax.experimental import pallas as pl
from jax.experimental.pallas import tpu as pltpu
from jax.experimental.pallas import tpu_sc as plsc

N = 10000
D = 128
E = 320000

NC = 2
NS = 16
NW = NC * NS
RPT = 320
NPAD = NW * RPT
K = 128
BLK = 2048

_sc_mesh = plsc.VectorSubcoreMesh(core_axis_name="c", subcore_axis_name="s")


def _segsum_body(x_hbm, src_hbm, dst_hbm, offs_hbm, out_hbm,
                 agg_v, srcb, dstb, rows, offs_v, gsem):
    wid = lax.axis_index("c") * NS + lax.axis_index("s")
    row_base = wid * RPT
    pltpu.sync_copy(offs_hbm, offs_v.at[pl.ds(0, 48)])
    ov = offs_v[pl.ds(wid, 16)]
    e0 = ov[0]
    e1 = ov[1]

    @pl.loop(0, RPT * D, step=128)
    def _zero(i):
        z = jnp.zeros((16,), jnp.float32)
        for c in range(8):
            agg_v[pl.ds(i + c * 16, 16)] = z

    a0 = (e0 // K) * K
    nch = jnp.maximum((e1 - a0 + K - 1) // K, 0)

    @pl.loop(0, nch)
    def _chunk(ci):
        a = a0 + ci * K
        pltpu.sync_copy(src_hbm.at[pl.ds(a, K)], srcb)
        pltpu.sync_copy(dst_hbm.at[pl.ds(a, K)], dstb.at[pl.ds(0, K)])
        pltpu.async_copy(x_hbm.at[srcb], rows, gsem).wait()

        @pl.loop(0, K)
        def _edge(j):
            g = a + j

            @pl.when(jnp.logical_and(g >= e0, g < e1))
            def _():
                r = dstb[pl.ds(j, 16)][0] - row_base
                base = r * D
                for c in range(8):
                    vec = rows[j, pl.ds(c * 16, 16)]
                    plsc.addupdate(agg_v.at[pl.ds(base + c * 16, 16)], vec)

    pltpu.sync_copy(agg_v, out_hbm.at[pl.ds(row_base * D, RPT * D)])


@jax.jit
def _segsum_sc(xp, src_p, dst_p, offs):
    return pl.kernel(
        _segsum_body,
        out_type=jax.ShapeDtypeStruct((NPAD * D,), jnp.float32),
        mesh=_sc_mesh,
        scratch_types=[
            pltpu.VMEM((RPT * D,), jnp.float32),
            pltpu.VMEM((K,), jnp.int32),
            pltpu.VMEM((K + 16,), jnp.int32),
            pltpu.VMEM((K, D), jnp.float32),
            pltpu.VMEM((64,), jnp.int32),
            pltpu.SemaphoreType.DMA,
        ],
    )(xp, src_p, dst_p, offs).reshape(NPAD, D)


def _dense_body(x_ref, agg_ref, ws_ref, wn_ref, b_ref, o_ref, *, mode):
    acc = jnp.dot(x_ref[...], ws_ref[...], preferred_element_type=jnp.float32)
    acc += jnp.dot(agg_ref[...], wn_ref[...], preferred_element_type=jnp.float32)
    acc += b_ref[...]
    if mode == "relu":
        o_ref[...] = jnp.maximum(acc, 0.0)
    else:
        o_ref[...] = acc


def _dense_res_body(x_ref, agg_ref, ws_ref, wn_ref, b_ref, temp_ref, o_ref):
    acc = jnp.dot(x_ref[...], ws_ref[...], preferred_element_type=jnp.float32)
    acc += jnp.dot(agg_ref[...], wn_ref[...], preferred_element_type=jnp.float32)
    acc += b_ref[...]
    o_ref[...] = (temp_ref[...] + jnp.maximum(acc, 0.0)) * 0.5


def _row_spec():
    return pl.BlockSpec((BLK, D), lambda i: (i, 0))


def _w_spec():
    return pl.BlockSpec((D, D), lambda i: (0, 0))


def _b_spec():
    return pl.BlockSpec((1, D), lambda i: (0, 0))


@functools.partial(jax.jit, static_argnames=("mode",))
def _dense_layer(x, agg, ws, wn, b, mode):
    return pl.pallas_call(
        functools.partial(_dense_body, mode=mode),
        grid=(NPAD // BLK,),
        in_specs=[_row_spec(), _row_spec(), _w_spec(), _w_spec(), _b_spec()],
        out_specs=_row_spec(),
        out_shape=jax.ShapeDtypeStruct((NPAD, D), jnp.float32),
    )(x, agg, ws, wn, b.reshape(1, D))


@jax.jit
def _dense_res_layer(x, agg, ws, wn, b, temp):
    return pl.pallas_call(
        _dense_res_body,
        grid=(NPAD // BLK,),
        in_specs=[_row_spec(), _row_spec(), _w_spec(), _w_spec(), _b_spec(),
                  _row_spec()],
        out_specs=_row_spec(),
        out_shape=jax.ShapeDtypeStruct((NPAD, D), jnp.float32),
    )(x, agg, ws, wn, b.reshape(1, D), temp)


def kernel(edges, shape_features, Ws_self, Ws_neigh, bs, Wf_self, Wf_neigh, bf):
    src = edges[0]
    dst = edges[1]

    dst_s, src_s = lax.sort((dst, src), num_keys=1)
    src_p = jnp.concatenate([src_s, jnp.zeros((K,), jnp.int32)])
    dst_p = jnp.concatenate([dst_s, jnp.zeros((K,), jnp.int32)])
    bounds = jnp.arange(33, dtype=jnp.int32) * RPT
    offs = jnp.searchsorted(dst_s, bounds).astype(jnp.int32)
    offs = jnp.concatenate([offs, jnp.zeros((15,), jnp.int32)])

    xp = jnp.zeros((NPAD, D), jnp.float32).at[:N].set(shape_features)

    x = _dense_layer(xp, _segsum_sc(xp, src_p, dst_p, offs),
                     Ws_self[0], Ws_neigh[0], bs[0], mode="relu")
    for i in range(1, 12, 2):
        temp = x
        x = _dense_layer(x, _segsum_sc(x, src_p, dst_p, offs),
                         Ws_self[i], Ws_neigh[i], bs[i], mode="relu")
        x = _dense_res_layer(x, _segsum_sc(x, src_p, dst_p, offs),
                             Ws_self[i + 1], Ws_neigh[i + 1], bs[i + 1], temp)

    wf_s = jnp.zeros((D, D), jnp.float32).at[:, :3].set(Wf_self)
    wf_n = jnp.zeros((D, D), jnp.float32).at[:, :3].set(Wf_neigh)
    bf_p = jnp.zeros((D,), jnp.float32).at[:3].set(bf)
    coords_p = _dense_layer(x, _segsum_sc(x, src_p, dst_p, offs),
                            wf_s, wf_n, bf_p, mode="linear")
    return (x[:N], coords_p[:N, :3])


# speedup vs baseline: 4.4339x; 1.7659x over previous
import functools

import jax
import jax.numpy as jnp
from jax import lax
from jax.experimental import pallas as pl
from jax.experimental.pallas import tpu as pltpu
from jax.experimental.pallas import tpu_sc as plsc

N = 10000
D = 128
E = 320000

NC = 2
NS = 16
NW = NC * NS
RPT = 320
NPAD = NW * RPT
K = 128
CPS = 16
SUP = K * CPS
EPAD = E + SUP
BLK = 2048

_sc_mesh = plsc.VectorSubcoreMesh(core_axis_name="c", subcore_axis_name="s")


def _segsum_body(x_hbm, src_hbm, dst_hbm, offs_hbm, out_hbm,
                 agg_v, srcb0, srcb1, dstb0, dstb1, rows0, rows1, offs_v,
                 isem0, isem1, gsem0, gsem1):
    wid = lax.axis_index("c") * NS + lax.axis_index("s")
    row_base = wid * RPT
    pltpu.sync_copy(offs_hbm, offs_v.at[pl.ds(0, 48)])
    ov = offs_v[pl.ds(wid, 16)]
    e0 = ov[0]
    e1 = ov[1]

    @pl.loop(0, (RPT + 1) * D, step=128)
    def _zero(i):
        z = jnp.zeros((16,), jnp.float32)
        for c in range(8):
            agg_v[pl.ds(i + c * 16, 16)] = z

    c0 = e0 // K
    c1 = (e1 + K - 1) // K
    nch = jnp.maximum(c1 - c0, 0)
    nsup = (nch + CPS - 1) // CPS

    srcbs = (srcb0, srcb1)
    dstbs = (dstb0, dstb1)
    rows = (rows0, rows1)
    isems = (isem0, isem1)
    gsems = (gsem0, gsem1)

    def idx_copies(s, sl):
        start = (c0 + s * CPS) * K
        return (pltpu.make_async_copy(src_hbm.at[pl.ds(start, SUP)],
                                      srcbs[sl], isems[sl]),
                pltpu.make_async_copy(dst_hbm.at[pl.ds(start, SUP)],
                                      dstbs[sl], isems[sl]))

    def gather(sl, kk, gl):
        return pltpu.make_async_copy(
            x_hbm.at[srcbs[sl].at[pl.ds(kk * K, K)]], rows[gl], gsems[gl])

    def process_chunk(sl, kk, gl):
        rbuf = rows[gl]
        dbuf = dstbs[sl]

        @pl.loop(0, K, step=16)
        def _grp(j0):
            dv = dbuf[pl.ds(kk * K + j0, 16)]
            rv = dv - row_base
            ok = jnp.logical_and(rv >= 0, rv < RPT)
            bases = jnp.where(ok, rv, RPT) * D
            for l in range(16):
                b = bases[l]
                for c in range(8):
                    vec = rbuf[j0 + l, pl.ds(c * 16, 16)]
                    plsc.addupdate(agg_v.at[pl.ds(b + c * 16, 16)], vec)

    @pl.when(nsup > 0)
    def _():
        for cp in idx_copies(0, 0):
            cp.start()

    @pl.loop(0, nsup, step=2)
    def _super(sb):
        for sl in (0, 1):
            s = sb + sl

            @pl.when(s < nsup)
            def _():
                cs0 = c0 + s * CPS
                ng = jnp.minimum(c1 - cs0, CPS)
                for cp in idx_copies(s, sl):
                    cp.wait()

                @pl.when(s + 1 < nsup)
                def _():
                    for cp in idx_copies(s + 1, 1 - sl):
                        cp.start()

                @pl.when(ng > 0)
                def _():
                    gather(sl, 0, 0).start()

                @pl.when(ng > 1)
                def _():
                    gather(sl, 1, 1).start()

                @pl.loop(0, CPS, step=2)
                def _chunks(kb):
                    for gl in (0, 1):
                        kk = kb + gl

                        @pl.when(kk < ng)
                        def _():
                            gather(sl, kk, gl).wait()
                            process_chunk(sl, kk, gl)

                            @pl.when(kk + 2 < ng)
                            def _():
                                gather(sl, kk + 2, gl).start()

    pltpu.sync_copy(agg_v.at[pl.ds(0, RPT * D)],
                    out_hbm.at[pl.ds(row_base * D, RPT * D)])


@jax.jit
def _segsum_sc(xp, src_p, dst_p, offs):
    return pl.kernel(
        _segsum_body,
        out_type=jax.ShapeDtypeStruct((NPAD * D,), jnp.float32),
        mesh=_sc_mesh,
        scratch_types=[
            pltpu.VMEM(((RPT + 1) * D,), jnp.float32),
            pltpu.VMEM((SUP,), jnp.int32),
            pltpu.VMEM((SUP,), jnp.int32),
            pltpu.VMEM((SUP,), jnp.int32),
            pltpu.VMEM((SUP,), jnp.int32),
            pltpu.VMEM((K, D), jnp.float32),
            pltpu.VMEM((K, D), jnp.float32),
            pltpu.VMEM((64,), jnp.int32),
            pltpu.SemaphoreType.DMA,
            pltpu.SemaphoreType.DMA,
            pltpu.SemaphoreType.DMA,
            pltpu.SemaphoreType.DMA,
        ],
    )(xp, src_p, dst_p, offs).reshape(NPAD, D)


def _dense_body(x_ref, agg_ref, ws_ref, wn_ref, b_ref, o_ref, *, mode):
    acc = jnp.dot(x_ref[...], ws_ref[...], preferred_element_type=jnp.float32)
    acc += jnp.dot(agg_ref[...], wn_ref[...], preferred_element_type=jnp.float32)
    acc += b_ref[...]
    if mode == "relu":
        o_ref[...] = jnp.maximum(acc, 0.0)
    else:
        o_ref[...] = acc


def _dense_res_body(x_ref, agg_ref, ws_ref, wn_ref, b_ref, temp_ref, o_ref):
    acc = jnp.dot(x_ref[...], ws_ref[...], preferred_element_type=jnp.float32)
    acc += jnp.dot(agg_ref[...], wn_ref[...], preferred_element_type=jnp.float32)
    acc += b_ref[...]
    o_ref[...] = (temp_ref[...] + jnp.maximum(acc, 0.0)) * 0.5


def _row_spec():
    return pl.BlockSpec((BLK, D), lambda i: (i, 0))


def _w_spec():
    return pl.BlockSpec((D, D), lambda i: (0, 0))


def _b_spec():
    return pl.BlockSpec((1, D), lambda i: (0, 0))


@functools.partial(jax.jit, static_argnames=("mode",))
def _dense_layer(x, agg, ws, wn, b, mode):
    return pl.pallas_call(
        functools.partial(_dense_body, mode=mode),
        grid=(NPAD // BLK,),
        in_specs=[_row_spec(), _row_spec(), _w_spec(), _w_spec(), _b_spec()],
        out_specs=_row_spec(),
        out_shape=jax.ShapeDtypeStruct((NPAD, D), jnp.float32),
    )(x, agg, ws, wn, b.reshape(1, D))


@jax.jit
def _dense_res_layer(x, agg, ws, wn, b, temp):
    return pl.pallas_call(
        _dense_res_body,
        grid=(NPAD // BLK,),
        in_specs=[_row_spec(), _row_spec(), _w_spec(), _w_spec(), _b_spec(),
                  _row_spec()],
        out_specs=_row_spec(),
        out_shape=jax.ShapeDtypeStruct((NPAD, D), jnp.float32),
    )(x, agg, ws, wn, b.reshape(1, D), temp)


def kernel(edges, shape_features, Ws_self, Ws_neigh, bs, Wf_self, Wf_neigh, bf):
    src = edges[0]
    dst = edges[1]

    dst_s, src_s = lax.sort((dst, src), num_keys=1)
    src_p = jnp.concatenate([src_s, jnp.zeros((EPAD - E,), jnp.int32)])
    dst_p = jnp.concatenate([dst_s, jnp.full((EPAD - E,), NPAD, jnp.int32)])
    bounds = jnp.arange(33, dtype=jnp.int32) * RPT
    offs = jnp.searchsorted(dst_s, bounds).astype(jnp.int32)
    offs = jnp.concatenate([offs, jnp.zeros((15,), jnp.int32)])

    xp = jnp.zeros((NPAD, D), jnp.float32).at[:N].set(shape_features)

    x = _dense_layer(xp, _segsum_sc(xp, src_p, dst_p, offs),
                     Ws_self[0], Ws_neigh[0], bs[0], mode="relu")
    for i in range(1, 12, 2):
        temp = x
        x = _dense_layer(x, _segsum_sc(x, src_p, dst_p, offs),
                         Ws_self[i], Ws_neigh[i], bs[i], mode="relu")
        x = _dense_res_layer(x, _segsum_sc(x, src_p, dst_p, offs),
                             Ws_self[i + 1], Ws_neigh[i + 1], bs[i + 1], temp)

    wf_s = jnp.zeros((D, D), jnp.float32).at[:, :3].set(Wf_self)
    wf_n = jnp.zeros((D, D), jnp.float32).at[:, :3].set(Wf_neigh)
    bf_p = jnp.zeros((D,), jnp.float32).at[:3].set(bf)
    coords_p = _dense_layer(x, _segsum_sc(x, src_p, dst_p, offs),
                            wf_s, wf_n, bf_p, mode="linear")
    return (x[:N], coords_p[:N, :3])
